# Initial kernel scaffold; baseline (speedup 1.0000x reference)
import jax
import jax.numpy as jnp
from jax.experimental import pallas as pl


def kernel(x, edge_index, edge_attr, W1, b1, W2, b2, W3, b3, W4, b4):
    raise NotImplementedError("write your pallas kernel here")


# speedup vs baseline: 14.6735x; 14.6735x over previous
import functools

import jax
import jax.numpy as jnp
from jax import lax
from jax.experimental import pallas as pl
from jax.experimental.pallas import tpu as pltpu
from jax.experimental.pallas import tpu_sc as plsc

_N = 10000
_NP = 10240
_D = 128
_NC = 2
_NS = 16
_NW = _NC * _NS
_CHUNK = 128
_E = 320000
_CPT = 80
_EPT = _CPT * _CHUNK
_EPAD = _NW * _EPT
_RPT = _NP // _NS


def _row_pass_body(src_hbm, dst_hbm, w_hbm, g_hbm, out_hbm,
                   src_v, dst_v, w_v, rows_v, acc_sh, sem):
    c = lax.axis_index("c")
    s = lax.axis_index("s")
    wid = c * _NS + s
    pltpu.sync_copy(src_hbm.at[pl.ds(wid * _CPT, _CPT)], src_v)
    pltpu.sync_copy(dst_hbm.at[pl.ds(wid * _CPT, _CPT)], dst_v)
    pltpu.sync_copy(w_hbm.at[pl.ds(wid * _CPT, _CPT)], w_v)

    zero16 = jnp.zeros((16,), jnp.float32)

    def _zero_row(r, carry):
        for k in range(8):
            rows_v[r, pl.ds(k * 16, 16)] = zero16
        return carry

    lax.fori_loop(0, _CHUNK, _zero_row, 0)
    base = s * _RPT
    for k in range(_RPT // _CHUNK):
        pltpu.sync_copy(rows_v, acc_sh.at[pl.ds(base + k * _CHUNK, _CHUNK)])
    plsc.subcore_barrier()

    def _chunk(j, carry):
        pltpu.async_copy(g_hbm.at[src_v.at[j]], rows_v, sem).wait()

        def _scale(m, carry2):
            wv = w_v[j, pl.ds(m * 16, 16)]
            dnums = lax.GatherDimensionNumbers(
                offset_dims=(), collapsed_slice_dims=(0,),
                start_index_map=(0,))
            for t in range(16):
                wr = lax.gather(wv, jnp.full((16, 1), t, jnp.int32), dnums,
                                (1,),
                                mode=lax.GatherScatterMode.PROMISE_IN_BOUNDS)
                r = m * 16 + t
                for k in range(8):
                    sl = pl.ds(k * 16, 16)
                    rows_v[r, sl] = rows_v[r, sl] * wr
            return carry2

        lax.fori_loop(0, _CHUNK // 16, _scale, 0)
        pltpu.sync_copy(rows_v, acc_sh.at[dst_v.at[j]], add=True)
        return carry

    lax.fori_loop(0, _CPT, _chunk, 0)
    plsc.subcore_barrier()
    pltpu.sync_copy(acc_sh.at[pl.ds(base, _RPT)],
                    out_hbm.at[c, pl.ds(base, _RPT)])


_row_pass = functools.partial(
    pl.kernel,
    out_type=jax.ShapeDtypeStruct((_NC, _NP, _D), jnp.float32),
    mesh=plsc.VectorSubcoreMesh(core_axis_name="c", subcore_axis_name="s"),
    scratch_types=[
        pltpu.VMEM((_CPT, _CHUNK), jnp.int32),
        pltpu.VMEM((_CPT, _CHUNK), jnp.int32),
        pltpu.VMEM((_CPT, _CHUNK), jnp.float32),
        pltpu.VMEM((_CHUNK, _D), jnp.float32),
        pltpu.VMEM_SHARED((_NP, _D), jnp.float32),
        pltpu.SemaphoreType.DMA,
    ],
)(_row_pass_body)


def _scalar_pass_body(src_hbm, dst_hbm, w_hbm, g_hbm, out_hbm,
                      src_v, dst_v, w_v, prod_v, zrow_v, acc_sh, sem):
    c = lax.axis_index("c")
    s = lax.axis_index("s")
    wid = c * _NS + s
    pltpu.sync_copy(src_hbm.at[pl.ds(wid * _CPT, _CPT)], src_v)
    pltpu.sync_copy(dst_hbm.at[pl.ds(wid * _CPT, _CPT)], dst_v)
    pltpu.sync_copy(w_hbm.at[pl.ds(wid * _CPT, _CPT)], w_v)

    zero16 = jnp.zeros((16,), jnp.float32)

    def _zero(i, carry):
        zrow_v[pl.ds(i * 16, 16)] = zero16
        return carry

    lax.fori_loop(0, _RPT // 16, _zero, 0)
    pltpu.sync_copy(zrow_v, acc_sh.at[pl.ds(s * _RPT, _RPT)])
    plsc.subcore_barrier()

    def _chunk(j, carry):
        pltpu.async_copy(g_hbm.at[src_v.at[j]], prod_v.at[j], sem).wait()
        for k in range(8):
            sl = pl.ds(k * 16, 16)
            prod_v[j, sl] = prod_v[j, sl] * w_v[j, sl]
        pltpu.sync_copy(prod_v.at[j], acc_sh.at[dst_v.at[j]], add=True)
        return carry

    lax.fori_loop(0, _CPT, _chunk, 0)
    plsc.subcore_barrier()
    pltpu.sync_copy(acc_sh.at[pl.ds(s * _RPT, _RPT)],
                    out_hbm.at[pl.ds(c * _NP + s * _RPT, _RPT)])


_scalar_pass = functools.partial(
    pl.kernel,
    out_type=jax.ShapeDtypeStruct((_NC * _NP,), jnp.float32),
    mesh=plsc.VectorSubcoreMesh(core_axis_name="c", subcore_axis_name="s"),
    scratch_types=[
        pltpu.VMEM((_CPT, _CHUNK), jnp.int32),
        pltpu.VMEM((_CPT, _CHUNK), jnp.int32),
        pltpu.VMEM((_CPT, _CHUNK), jnp.float32),
        pltpu.VMEM((_CPT, _CHUNK), jnp.float32),
        pltpu.VMEM((_RPT,), jnp.float32),
        pltpu.VMEM_SHARED((_NP,), jnp.float32),
        pltpu.SemaphoreType.DMA,
    ],
)(_scalar_pass_body)


def _dinv_body(degp_ref, dinv_ref):
    deg = degp_ref[0:1, :_N] + degp_ref[1:2, :_N] + 1.0
    dinv_ref[...] = jnp.where(deg > 0.0, lax.rsqrt(deg), 0.0)


def _dinv_pass(degp):
    return pl.pallas_call(
        _dinv_body,
        out_shape=jax.ShapeDtypeStruct((1, _N), jnp.float32),
    )(degp)


def _mm_body(h_ref, w_ref, dinv_ref, g_ref):
    g_ref[...] = dinv_ref[...] * jnp.dot(
        h_ref[...], w_ref[...], preferred_element_type=jnp.float32)


def _mm_pass(h, w, dinv_col):
    return pl.pallas_call(
        _mm_body,
        out_shape=jax.ShapeDtypeStruct((_N, w.shape[1]), jnp.float32),
    )(h, w, dinv_col)


def _comb_body(sp_ref, g_ref, dinv_ref, b_ref, wn_ref, out_ref):
    ssum = sp_ref[0, :_N, :] + sp_ref[1, :_N, :] + g_ref[...]
    h = jnp.maximum(dinv_ref[...] * ssum + b_ref[...], 0.0)
    out_ref[...] = dinv_ref[...] * jnp.dot(
        h, wn_ref[...], preferred_element_type=jnp.float32)


def _comb_pass(sp, g, dinv_col, b, wn):
    return pl.pallas_call(
        _comb_body,
        out_shape=jax.ShapeDtypeStruct((_N, wn.shape[1]), jnp.float32),
    )(sp, g, dinv_col, b, wn)


def _fin_body(s4p_ref, g4_ref, dinv_ref, b4_ref, out_ref):
    s4 = s4p_ref[0:1, :_N] + s4p_ref[1:2, :_N] + g4_ref[...]
    out_ref[...] = dinv_ref[...] * s4 + b4_ref[...]


def _fin_pass(s4p, g4_row, dinv_row, b4):
    return pl.pallas_call(
        _fin_body,
        out_shape=jax.ShapeDtypeStruct((1, _N), jnp.float32),
    )(s4p, g4_row, dinv_row, b4)


def kernel(x, edge_index, edge_attr, W1, b1, W2, b2, W3, b3, W4, b4):
    src = edge_index[0]
    dst = edge_index[1]
    ew = edge_attr[:, 0]
    pad = _EPAD - _E
    pidx = jnp.arange(pad, dtype=jnp.int32) % _N
    srcp = jnp.concatenate([src, pidx]).reshape(_NW * _CPT, _CHUNK)
    dstp = jnp.concatenate([dst, pidx]).reshape(_NW * _CPT, _CHUNK)
    wp = jnp.concatenate(
        [ew, jnp.zeros((pad,), jnp.float32)]).reshape(_NW * _CPT, _CHUNK)

    ones_t = jnp.ones((_N,), jnp.float32)
    degp = _scalar_pass(srcp, dstp, wp, ones_t).reshape(_NC, _NP)
    dinv_row = _dinv_pass(degp)
    dinv_col = dinv_row.reshape(_N, 1)

    g1 = _mm_pass(x, W1, dinv_col)
    s1 = _row_pass(srcp, dstp, wp, g1)
    g2 = _comb_pass(s1, g1, dinv_col, b1.reshape(1, _D), W2)
    s2 = _row_pass(srcp, dstp, wp, g2)
    g3 = _comb_pass(s2, g2, dinv_col, b2.reshape(1, _D), W3)
    s3 = _row_pass(srcp, dstp, wp, g3)
    g4 = _comb_pass(s3, g3, dinv_col, b3.reshape(1, _D), W4)
    g4_flat = g4.reshape(_N)
    s4 = _scalar_pass(srcp, dstp, wp, g4_flat).reshape(_NC, _NP)
    out_row = _fin_pass(s4, g4.reshape(1, _N), dinv_row, b4.reshape(1, 1))
    return out_row.reshape(_N, 1)
